# baseline (speedup 1.0000x reference)
import functools

import jax
import jax.numpy as jnp
from jax import lax
from jax.experimental import pallas as pl
from jax.experimental.pallas import tpu as pltpu
from jax.experimental.pallas import tpu_sc as plsc

N = 10000
NP = 10240
E = 160000
EP = 163840
D = 16
T = 8
NC = 2
NS = 16
NW = NC * NS
WIN = 128
NWIN = EP // (NS * WIN)
ROWS_PER_TILE = NP // NW
ZCH = NP // NS

_mesh = plsc.VectorSubcoreMesh(core_axis_name="c", subcore_axis_name="s")
_sc_params = pltpu.CompilerParams(use_tc_tiling_on_sc=False)


@functools.partial(
    pl.kernel,
    out_type=jax.ShapeDtypeStruct((NP, D), jnp.float32),
    mesh=_mesh,
    scratch_types=[
        pltpu.VMEM((4, 80), jnp.int32),
        pltpu.VMEM((80, D), jnp.float32),
    ],
    compiler_params=_sc_params,
)
def _sc_embed(emb_hbm, tok_hbm, out_hbm, idx_v, rows_v):
    c = lax.axis_index("c")
    s = lax.axis_index("s")
    wid = c * NS + s
    pltpu.sync_copy(tok_hbm.at[wid], idx_v)

    @pl.loop(0, 4)
    def _(k):
        pltpu.sync_copy(emb_hbm.at[idx_v.at[k]], rows_v)
        pltpu.sync_copy(rows_v, out_hbm.at[pl.ds(wid * ROWS_PER_TILE + k * 80, 80)])


@functools.partial(
    pl.kernel,
    out_type=jax.ShapeDtypeStruct((NC, NP, D), jnp.float32),
    mesh=_mesh,
    scratch_types=[
        pltpu.VMEM((NWIN, WIN), jnp.int32),
        pltpu.VMEM((NWIN, WIN), jnp.int32),
        pltpu.VMEM((8, WIN, D), jnp.float32),
        pltpu.VMEM((WIN, D), jnp.float32),
        pltpu.VMEM_SHARED((NP, D), jnp.float32),
        pltpu.SemaphoreType.DMA((8,)),
        pltpu.SemaphoreType.DMA((2,)),
    ],
    compiler_params=_sc_params,
)
def _sc_pass(tab, gi_hbm, si_hbm, outP, gv, sv, rows, zv, acc, semG, semI):
    c = lax.axis_index("c")
    s = lax.axis_index("s")
    on = c == 0

    @pl.when(on)
    def _():
        pltpu.async_copy(gi_hbm.at[s], gv, semI.at[0])
        pltpu.async_copy(si_hbm.at[s], sv, semI.at[1])

    @pl.loop(0, WIN)
    def _(i):
        zv[i, :] = jnp.zeros((D,), jnp.float32)

    @pl.loop(0, ZCH // WIN)
    def _(k):
        pltpu.sync_copy(zv, acc.at[pl.ds(s * ZCH + k * WIN, WIN)])

    plsc.subcore_barrier()

    @pl.when(on)
    def _():
        pltpu.make_async_copy(gi_hbm.at[s], gv, semI.at[0]).wait()
        pltpu.make_async_copy(si_hbm.at[s], sv, semI.at[1]).wait()

        for k in range(8):
            pltpu.async_copy(tab.at[gv.at[k]], rows.at[k], semG.at[k])

        @pl.loop(0, NWIN // 8)
        def _(i):
            for k in range(8):
                w = 8 * i + k
                pltpu.make_async_copy(tab.at[gv.at[w]], rows.at[k], semG.at[k]).wait()
                pltpu.sync_copy(rows.at[k], acc.at[sv.at[w]], add=True)

                @pl.when(w + 8 < NWIN)
                def _():
                    pltpu.async_copy(tab.at[gv.at[w + 8]], rows.at[k], semG.at[k])

    plsc.subcore_barrier()

    pltpu.sync_copy(acc.at[pl.ds(s * ZCH, ZCH)], outP.at[c, pl.ds(s * ZCH, ZCH)])


_R = 2048
_NPB = NP // _R


def _tables0_body(p_ref, wa_ref, ga_ref):
    ga_ref[...] = jnp.dot(p_ref[...], wa_ref[...], preferred_element_type=jnp.float32)


def _tc_tables(P, WA):
    return pl.pallas_call(
        _tables0_body,
        grid=(_NPB,),
        in_specs=[
            pl.BlockSpec((_R, D), lambda i: (i, 0)),
            pl.BlockSpec((D, T * D), lambda i: (0, 0)),
        ],
        out_specs=pl.BlockSpec((_R, T * D), lambda i: (i, 0)),
        out_shape=jax.ShapeDtypeStruct((NP, T * D), jnp.float32),
    )(P, WA)


def _gru_body(with_tables, p_ref, inp_ref, cnt_ref, wr_ref, wz_ref, wt_ref,
              w2_ref, b_ref, wb_ref, wa_ref, ex_ref, *out_refs):
    p = p_ref[...]
    infea = inp_ref[0] + inp_ref[1]
    cnt = cnt_ref[0] + cnt_ref[1]
    zb = jnp.dot(p, wb_ref[...], preferred_element_type=jnp.float32)
    owide = jnp.dot(cnt, ex_ref[...], preferred_element_type=jnp.float32) * zb

    def lin(w_ref, w2, x3, brow):
        w = w_ref[...]
        return (jnp.dot(infea, w[0:D], preferred_element_type=jnp.float32)
                + jnp.dot(owide, w2, preferred_element_type=jnp.float32)
                + jnp.dot(x3, w[2 * D:3 * D], preferred_element_type=jnp.float32)
                + brow)

    b = b_ref[...]
    w2 = w2_ref[...]
    r = jax.nn.sigmoid(lin(wr_ref, w2[0], p, b[0:1]))
    z = jax.nn.sigmoid(lin(wz_ref, w2[1], p, b[1:2]))
    h = jnp.tanh(lin(wt_ref, w2[2], r * p, b[2:3]))
    pnew = (1.0 - z) * p + z * h
    out_refs[0][...] = pnew
    if with_tables:
        out_refs[1][...] = jnp.dot(pnew, wa_ref[...], preferred_element_type=jnp.float32)


def _tc_gru(P, inP, cntP, Wr, Wz, Wt, W2, Bs, WB, WA, EX, with_tables):
    out_shape = [jax.ShapeDtypeStruct((NP, D), jnp.float32)]
    out_specs = [pl.BlockSpec((_R, D), lambda i: (i, 0))]
    if with_tables:
        out_shape.append(jax.ShapeDtypeStruct((NP, T * D), jnp.float32))
        out_specs.append(pl.BlockSpec((_R, T * D), lambda i: (i, 0)))
    return pl.pallas_call(
        functools.partial(_gru_body, with_tables),
        grid=(_NPB,),
        in_specs=[
            pl.BlockSpec((_R, D), lambda i: (i, 0)),
            pl.BlockSpec((NC, _R, D), lambda i: (0, i, 0)),
            pl.BlockSpec((NC, _R, D), lambda i: (0, i, 0)),
            pl.BlockSpec((3 * D, D), lambda i: (0, 0)),
            pl.BlockSpec((3 * D, D), lambda i: (0, 0)),
            pl.BlockSpec((3 * D, D), lambda i: (0, 0)),
            pl.BlockSpec((3, T * D, D), lambda i: (0, 0, 0)),
            pl.BlockSpec((3, D), lambda i: (0, 0)),
            pl.BlockSpec((D, T * D), lambda i: (0, 0)),
            pl.BlockSpec((D, T * D), lambda i: (0, 0)),
            pl.BlockSpec((D, T * D), lambda i: (0, 0)),
        ],
        out_specs=out_specs,
        out_shape=out_shape,
    )(P, inP, cntP, Wr, Wz, Wt, W2, Bs, WB, WA, EX)


def kernel(token, edge_index, edge_type, emb, fc_w, Wr, br, Wz, bz, Wt, bt):
    src = edge_index[0].astype(jnp.int32)
    dst = edge_index[1].astype(jnp.int32)
    et = edge_type.astype(jnp.int32)

    zpad = jnp.zeros((EP - E,), jnp.int32)
    npad = jnp.full((EP - E,), N, jnp.int32)
    ga = jnp.concatenate([src * T + et, zpad]).reshape(NS, NWIN, WIN)
    dsti = jnp.concatenate([dst, npad]).reshape(NS, NWIN, WIN)
    tile_id = jnp.arange(EP, dtype=jnp.int32) // (EP // NS)
    cg = jnp.concatenate([et, zpad]) + tile_id * T
    cgi = cg.reshape(NS, NWIN, WIN)
    srci = jnp.concatenate([src, npad]).reshape(NS, NWIN, WIN)

    tok = jnp.concatenate([token.astype(jnp.int32),
                           jnp.zeros((NP - N,), jnp.int32)]).reshape(NW, 4, 80)

    fw = fc_w.reshape(T, D, 2 * D)
    WA = fw[:, :, 0::2].transpose(1, 0, 2).reshape(D, T * D)
    WB = fw[:, :, 1::2].transpose(1, 0, 2).reshape(D, T * D)
    Bs = jnp.stack([br, bz, bt])
    OH = jnp.tile(jnp.eye(T, D, dtype=jnp.float32), (NS, 1))
    W2 = jnp.stack([jnp.tile(Wr[D:2 * D], (T, 1)),
                    jnp.tile(Wz[D:2 * D], (T, 1)),
                    jnp.tile(Wt[D:2 * D], (T, 1))])
    EX = jnp.concatenate([
        jnp.kron(jnp.eye(T, dtype=jnp.float32), jnp.ones((1, D), jnp.float32)),
        jnp.zeros((D - T, T * D), jnp.float32)])

    P = _sc_embed(emb, tok)
    cntP = _sc_pass(OH, cgi, srci)
    GA = _tc_tables(P, WA)
    for step in range(3):
        inP = _sc_pass(GA.reshape(NP * T, D), ga, dsti)
        last = step == 2
        res = _tc_gru(P, inP, cntP, Wr, Wz, Wt, W2, Bs, WB, WA, EX, not last)
        if last:
            P = res[0]
        else:
            P, GA = res
    return P[:N]


# speedup vs baseline: 1.0011x; 1.0011x over previous
import functools

import jax
import jax.numpy as jnp
from jax import lax
from jax.experimental import pallas as pl
from jax.experimental.pallas import tpu as pltpu
from jax.experimental.pallas import tpu_sc as plsc

N = 10000
NP = 10240
E = 160000
EP = 163840
D = 16
T = 8
NC = 2
NS = 16
NW = NC * NS
WIN = 128
NWIN = EP // (NS * WIN)
ROWS_PER_TILE = NP // NW
ZCH = NP // NS

_mesh = plsc.VectorSubcoreMesh(core_axis_name="c", subcore_axis_name="s")
_sc_params = pltpu.CompilerParams(use_tc_tiling_on_sc=False)


@functools.partial(
    pl.kernel,
    out_type=jax.ShapeDtypeStruct((NP, D), jnp.float32),
    mesh=_mesh,
    scratch_types=[
        pltpu.VMEM((4, 80), jnp.int32),
        pltpu.VMEM((80, D), jnp.float32),
    ],
    compiler_params=_sc_params,
)
def _sc_embed(emb_hbm, tok_hbm, out_hbm, idx_v, rows_v):
    c = lax.axis_index("c")
    s = lax.axis_index("s")
    wid = c * NS + s
    pltpu.sync_copy(tok_hbm.at[wid], idx_v)

    @pl.loop(0, 4)
    def _(k):
        pltpu.sync_copy(emb_hbm.at[idx_v.at[k]], rows_v)
        pltpu.sync_copy(rows_v, out_hbm.at[pl.ds(wid * ROWS_PER_TILE + k * 80, 80)])


@functools.partial(
    pl.kernel,
    out_type=jax.ShapeDtypeStruct((NC, NP, D), jnp.float32),
    mesh=_mesh,
    scratch_types=[
        pltpu.VMEM((NWIN, WIN), jnp.int32),
        pltpu.VMEM((NWIN, WIN), jnp.int32),
        pltpu.VMEM((8, WIN, D), jnp.float32),
        pltpu.VMEM((WIN, D), jnp.float32),
        pltpu.VMEM_SHARED((NP, D), jnp.float32),
        pltpu.SemaphoreType.DMA((8,)),
        pltpu.SemaphoreType.DMA((2,)),
    ],
    compiler_params=_sc_params,
)
def _sc_pass(tab, gi_hbm, si_hbm, outP, gv, sv, rows, zv, acc, semG, semI):
    c = lax.axis_index("c")
    s = lax.axis_index("s")
    on = c == 1

    @pl.when(on)
    def _():
        pltpu.async_copy(gi_hbm.at[s], gv, semI.at[0])
        pltpu.async_copy(si_hbm.at[s], sv, semI.at[1])

    @pl.loop(0, WIN)
    def _(i):
        zv[i, :] = jnp.zeros((D,), jnp.float32)

    @pl.loop(0, ZCH // WIN)
    def _(k):
        pltpu.sync_copy(zv, acc.at[pl.ds(s * ZCH + k * WIN, WIN)])

    plsc.subcore_barrier()

    @pl.when(on)
    def _():
        pltpu.make_async_copy(gi_hbm.at[s], gv, semI.at[0]).wait()
        pltpu.make_async_copy(si_hbm.at[s], sv, semI.at[1]).wait()

        for k in range(8):
            pltpu.async_copy(tab.at[gv.at[k]], rows.at[k], semG.at[k])

        @pl.loop(0, NWIN // 8)
        def _(i):
            for k in range(8):
                w = 8 * i + k
                pltpu.make_async_copy(tab.at[gv.at[w]], rows.at[k], semG.at[k]).wait()
                pltpu.sync_copy(rows.at[k], acc.at[sv.at[w]], add=True)

                @pl.when(w + 8 < NWIN)
                def _():
                    pltpu.async_copy(tab.at[gv.at[w + 8]], rows.at[k], semG.at[k])

    plsc.subcore_barrier()

    pltpu.sync_copy(acc.at[pl.ds(s * ZCH, ZCH)], outP.at[c, pl.ds(s * ZCH, ZCH)])


_R = 2048
_NPB = NP // _R


def _tables0_body(p_ref, wa_ref, ga_ref):
    ga_ref[...] = jnp.dot(p_ref[...], wa_ref[...], preferred_element_type=jnp.float32)


def _tc_tables(P, WA):
    return pl.pallas_call(
        _tables0_body,
        grid=(_NPB,),
        in_specs=[
            pl.BlockSpec((_R, D), lambda i: (i, 0)),
            pl.BlockSpec((D, T * D), lambda i: (0, 0)),
        ],
        out_specs=pl.BlockSpec((_R, T * D), lambda i: (i, 0)),
        out_shape=jax.ShapeDtypeStruct((NP, T * D), jnp.float32),
    )(P, WA)


def _gru_body(with_tables, p_ref, inp_ref, cnt_ref, wr_ref, wz_ref, wt_ref,
              w2_ref, b_ref, wb_ref, wa_ref, ex_ref, *out_refs):
    p = p_ref[...]
    infea = inp_ref[0] + inp_ref[1]
    cnt = cnt_ref[0] + cnt_ref[1]
    zb = jnp.dot(p, wb_ref[...], preferred_element_type=jnp.float32)
    owide = jnp.dot(cnt, ex_ref[...], preferred_element_type=jnp.float32) * zb

    def lin(w_ref, w2, x3, brow):
        w = w_ref[...]
        return (jnp.dot(infea, w[0:D], preferred_element_type=jnp.float32)
                + jnp.dot(owide, w2, preferred_element_type=jnp.float32)
                + jnp.dot(x3, w[2 * D:3 * D], preferred_element_type=jnp.float32)
                + brow)

    b = b_ref[...]
    w2 = w2_ref[...]
    r = jax.nn.sigmoid(lin(wr_ref, w2[0], p, b[0:1]))
    z = jax.nn.sigmoid(lin(wz_ref, w2[1], p, b[1:2]))
    h = jnp.tanh(lin(wt_ref, w2[2], r * p, b[2:3]))
    pnew = (1.0 - z) * p + z * h
    out_refs[0][...] = pnew
    if with_tables:
        out_refs[1][...] = jnp.dot(pnew, wa_ref[...], preferred_element_type=jnp.float32)


def _tc_gru(P, inP, cntP, Wr, Wz, Wt, W2, Bs, WB, WA, EX, with_tables):
    out_shape = [jax.ShapeDtypeStruct((NP, D), jnp.float32)]
    out_specs = [pl.BlockSpec((_R, D), lambda i: (i, 0))]
    if with_tables:
        out_shape.append(jax.ShapeDtypeStruct((NP, T * D), jnp.float32))
        out_specs.append(pl.BlockSpec((_R, T * D), lambda i: (i, 0)))
    return pl.pallas_call(
        functools.partial(_gru_body, with_tables),
        grid=(_NPB,),
        in_specs=[
            pl.BlockSpec((_R, D), lambda i: (i, 0)),
            pl.BlockSpec((NC, _R, D), lambda i: (0, i, 0)),
            pl.BlockSpec((NC, _R, D), lambda i: (0, i, 0)),
            pl.BlockSpec((3 * D, D), lambda i: (0, 0)),
            pl.BlockSpec((3 * D, D), lambda i: (0, 0)),
            pl.BlockSpec((3 * D, D), lambda i: (0, 0)),
            pl.BlockSpec((3, T * D, D), lambda i: (0, 0, 0)),
            pl.BlockSpec((3, D), lambda i: (0, 0)),
            pl.BlockSpec((D, T * D), lambda i: (0, 0)),
            pl.BlockSpec((D, T * D), lambda i: (0, 0)),
            pl.BlockSpec((D, T * D), lambda i: (0, 0)),
        ],
        out_specs=out_specs,
        out_shape=out_shape,
    )(P, inP, cntP, Wr, Wz, Wt, W2, Bs, WB, WA, EX)


def kernel(token, edge_index, edge_type, emb, fc_w, Wr, br, Wz, bz, Wt, bt):
    src = edge_index[0].astype(jnp.int32)
    dst = edge_index[1].astype(jnp.int32)
    et = edge_type.astype(jnp.int32)

    zpad = jnp.zeros((EP - E,), jnp.int32)
    npad = jnp.full((EP - E,), N, jnp.int32)
    ga = jnp.concatenate([src * T + et, zpad]).reshape(NS, NWIN, WIN)
    dsti = jnp.concatenate([dst, npad]).reshape(NS, NWIN, WIN)
    tile_id = jnp.arange(EP, dtype=jnp.int32) // (EP // NS)
    cg = jnp.concatenate([et, zpad]) + tile_id * T
    cgi = cg.reshape(NS, NWIN, WIN)
    srci = jnp.concatenate([src, npad]).reshape(NS, NWIN, WIN)

    tok = jnp.concatenate([token.astype(jnp.int32),
                           jnp.zeros((NP - N,), jnp.int32)]).reshape(NW, 4, 80)

    fw = fc_w.reshape(T, D, 2 * D)
    WA = fw[:, :, 0::2].transpose(1, 0, 2).reshape(D, T * D)
    WB = fw[:, :, 1::2].transpose(1, 0, 2).reshape(D, T * D)
    Bs = jnp.stack([br, bz, bt])
    OH = jnp.tile(jnp.eye(T, D, dtype=jnp.float32), (NS, 1))
    W2 = jnp.stack([jnp.tile(Wr[D:2 * D], (T, 1)),
                    jnp.tile(Wz[D:2 * D], (T, 1)),
                    jnp.tile(Wt[D:2 * D], (T, 1))])
    EX = jnp.concatenate([
        jnp.kron(jnp.eye(T, dtype=jnp.float32), jnp.ones((1, D), jnp.float32)),
        jnp.zeros((D - T, T * D), jnp.float32)])

    P = _sc_embed(emb, tok)
    cntP = _sc_pass(OH, cgi, srci)
    GA = _tc_tables(P, WA)
    for step in range(3):
        inP = _sc_pass(GA.reshape(NP * T, D), ga, dsti)
        last = step == 2
        res = _tc_gru(P, inP, cntP, Wr, Wz, Wt, W2, Bs, WB, WA, EX, not last)
        if last:
            P = res[0]
        else:
            P, GA = res
    return P[:N]


# speedup vs baseline: 1.1823x; 1.1810x over previous
import functools

import jax
import jax.numpy as jnp
from jax import lax
from jax.experimental import pallas as pl
from jax.experimental.pallas import tpu as pltpu
from jax.experimental.pallas import tpu_sc as plsc

N = 10000
NP = 10240
E = 160000
EP = 163840
D = 16
T = 8
NC = 2
NS = 16
NW = NC * NS
WIN = 128
NWIN = EP // (NW * WIN)
ROWS_PER_TILE = NP // NW
ZCH = NP // NS

_mesh = plsc.VectorSubcoreMesh(core_axis_name="c", subcore_axis_name="s")
_sc_params = pltpu.CompilerParams(use_tc_tiling_on_sc=False)


@functools.partial(
    pl.kernel,
    out_type=jax.ShapeDtypeStruct((NP, D), jnp.float32),
    mesh=_mesh,
    scratch_types=[
        pltpu.VMEM((4, 80), jnp.int32),
        pltpu.VMEM((80, D), jnp.float32),
    ],
    compiler_params=_sc_params,
)
def _sc_embed(emb_hbm, tok_hbm, out_hbm, idx_v, rows_v):
    c = lax.axis_index("c")
    s = lax.axis_index("s")
    wid = c * NS + s
    pltpu.sync_copy(tok_hbm.at[wid], idx_v)

    @pl.loop(0, 4)
    def _(k):
        pltpu.sync_copy(emb_hbm.at[idx_v.at[k]], rows_v)
        pltpu.sync_copy(rows_v, out_hbm.at[pl.ds(wid * ROWS_PER_TILE + k * 80, 80)])


@functools.partial(
    pl.kernel,
    out_type=jax.ShapeDtypeStruct((NC, NP, D), jnp.float32),
    mesh=_mesh,
    scratch_types=[
        pltpu.VMEM((NWIN, WIN), jnp.int32),
        pltpu.VMEM((NWIN, WIN), jnp.int32),
        pltpu.VMEM((8, WIN, D), jnp.float32),
        pltpu.VMEM((WIN, D), jnp.float32),
        pltpu.VMEM_SHARED((NP, D), jnp.float32),
        pltpu.SemaphoreType.DMA((8,)),
        pltpu.SemaphoreType.DMA((8,)),
        pltpu.SemaphoreType.DMA((2,)),
    ],
    compiler_params=_sc_params,
)
def _sc_pass(tab, gi_hbm, si_hbm, outP, gv, sv, rows, zv, acc, semG, semS, semI):
    c = lax.axis_index("c")
    s = lax.axis_index("s")
    wid = c * NS + s
    WBYTES = WIN * D * 4

    pltpu.async_copy(gi_hbm.at[wid], gv, semI.at[0])
    pltpu.async_copy(si_hbm.at[wid], sv, semI.at[1])

    @pl.loop(0, WIN)
    def _(i):
        zv[i, :] = jnp.zeros((D,), jnp.float32)

    @pl.loop(0, ZCH // WIN)
    def _(k):
        pltpu.sync_copy(zv, acc.at[pl.ds(s * ZCH + k * WIN, WIN)])

    pltpu.make_async_copy(gi_hbm.at[wid], gv, semI.at[0]).wait()
    pltpu.make_async_copy(si_hbm.at[wid], sv, semI.at[1]).wait()

    plsc.subcore_barrier()

    for k in range(4):
        pltpu.async_copy(tab.at[gv.at[k]], rows.at[k], semG.at[k])

    @pl.loop(0, NWIN // 8)
    def _(i):
        for k in range(8):
            w = 8 * i + k
            k4 = (k + 4) % 8
            pltpu.make_async_copy(tab.at[gv.at[w]], rows.at[k], semG.at[k]).wait()
            pltpu.async_copy(rows.at[k], acc.at[sv.at[w]], semS.at[k], add=True)

            @pl.when(w >= 4)
            def _():
                pltpu.make_async_copy(
                    tab.at[pl.ds(0, WIN)], rows.at[k4], semS.at[k4]).wait()

            @pl.when(w + 4 < NWIN)
            def _():
                pltpu.async_copy(tab.at[gv.at[w + 4]], rows.at[k4], semG.at[k4])

    for k in range(4, 8):
        pltpu.make_async_copy(tab.at[pl.ds(0, WIN)], rows.at[k], semS.at[k]).wait()

    plsc.subcore_barrier()

    pltpu.sync_copy(acc.at[pl.ds(s * ZCH, ZCH)], outP.at[c, pl.ds(s * ZCH, ZCH)])


_R = 2048
_NPB = NP // _R


def _tables0_body(p_ref, wa_ref, ga_ref):
    ga_ref[...] = jnp.dot(p_ref[...], wa_ref[...], preferred_element_type=jnp.float32)


def _tc_tables(P, WA):
    return pl.pallas_call(
        _tables0_body,
        grid=(_NPB,),
        in_specs=[
            pl.BlockSpec((_R, D), lambda i: (i, 0)),
            pl.BlockSpec((D, T * D), lambda i: (0, 0)),
        ],
        out_specs=pl.BlockSpec((_R, T * D), lambda i: (i, 0)),
        out_shape=jax.ShapeDtypeStruct((NP, T * D), jnp.float32),
    )(P, WA)


def _gru_body(with_tables, p_ref, inp_ref, cnt_ref, wr_ref, wz_ref, wt_ref,
              w2_ref, b_ref, wb_ref, wa_ref, ex_ref, *out_refs):
    p = p_ref[...]
    infea = inp_ref[0] + inp_ref[1]
    cnt = cnt_ref[0] + cnt_ref[1]
    zb = jnp.dot(p, wb_ref[...], preferred_element_type=jnp.float32)
    owide = jnp.dot(cnt, ex_ref[...], preferred_element_type=jnp.float32) * zb

    def lin(w_ref, w2, x3, brow):
        w = w_ref[...]
        return (jnp.dot(infea, w[0:D], preferred_element_type=jnp.float32)
                + jnp.dot(owide, w2, preferred_element_type=jnp.float32)
                + jnp.dot(x3, w[2 * D:3 * D], preferred_element_type=jnp.float32)
                + brow)

    b = b_ref[...]
    w2 = w2_ref[...]
    r = jax.nn.sigmoid(lin(wr_ref, w2[0], p, b[0:1]))
    z = jax.nn.sigmoid(lin(wz_ref, w2[1], p, b[1:2]))
    h = jnp.tanh(lin(wt_ref, w2[2], r * p, b[2:3]))
    pnew = (1.0 - z) * p + z * h
    out_refs[0][...] = pnew
    if with_tables:
        out_refs[1][...] = jnp.dot(pnew, wa_ref[...], preferred_element_type=jnp.float32)


def _tc_gru(P, inP, cntP, Wr, Wz, Wt, W2, Bs, WB, WA, EX, with_tables):
    out_shape = [jax.ShapeDtypeStruct((NP, D), jnp.float32)]
    out_specs = [pl.BlockSpec((_R, D), lambda i: (i, 0))]
    if with_tables:
        out_shape.append(jax.ShapeDtypeStruct((NP, T * D), jnp.float32))
        out_specs.append(pl.BlockSpec((_R, T * D), lambda i: (i, 0)))
    return pl.pallas_call(
        functools.partial(_gru_body, with_tables),
        grid=(_NPB,),
        in_specs=[
            pl.BlockSpec((_R, D), lambda i: (i, 0)),
            pl.BlockSpec((NC, _R, D), lambda i: (0, i, 0)),
            pl.BlockSpec((NC, _R, D), lambda i: (0, i, 0)),
            pl.BlockSpec((3 * D, D), lambda i: (0, 0)),
            pl.BlockSpec((3 * D, D), lambda i: (0, 0)),
            pl.BlockSpec((3 * D, D), lambda i: (0, 0)),
            pl.BlockSpec((3, T * D, D), lambda i: (0, 0, 0)),
            pl.BlockSpec((3, D), lambda i: (0, 0)),
            pl.BlockSpec((D, T * D), lambda i: (0, 0)),
            pl.BlockSpec((D, T * D), lambda i: (0, 0)),
            pl.BlockSpec((D, T * D), lambda i: (0, 0)),
        ],
        out_specs=out_specs,
        out_shape=out_shape,
    )(P, inP, cntP, Wr, Wz, Wt, W2, Bs, WB, WA, EX)


def kernel(token, edge_index, edge_type, emb, fc_w, Wr, br, Wz, bz, Wt, bt):
    src = edge_index[0].astype(jnp.int32)
    dst = edge_index[1].astype(jnp.int32)
    et = edge_type.astype(jnp.int32)

    zpad = jnp.zeros((EP - E,), jnp.int32)
    npad = jnp.full((EP - E,), N, jnp.int32)
    ga = jnp.concatenate([src * T + et, zpad]).reshape(NW, NWIN, WIN)
    dsti = jnp.concatenate([dst, npad]).reshape(NW, NWIN, WIN)
    tile_id = jnp.arange(EP, dtype=jnp.int32) // (EP // NW)
    cg = jnp.concatenate([et, zpad]) + tile_id * T
    cgi = cg.reshape(NW, NWIN, WIN)
    srci = jnp.concatenate([src, npad]).reshape(NW, NWIN, WIN)

    tok = jnp.concatenate([token.astype(jnp.int32),
                           jnp.zeros((NP - N,), jnp.int32)]).reshape(NW, 4, 80)

    fw = fc_w.reshape(T, D, 2 * D)
    WA = fw[:, :, 0::2].transpose(1, 0, 2).reshape(D, T * D)
    WB = fw[:, :, 1::2].transpose(1, 0, 2).reshape(D, T * D)
    Bs = jnp.stack([br, bz, bt])
    OH = jnp.tile(jnp.eye(T, D, dtype=jnp.float32), (NW, 1))
    W2 = jnp.stack([jnp.tile(Wr[D:2 * D], (T, 1)),
                    jnp.tile(Wz[D:2 * D], (T, 1)),
                    jnp.tile(Wt[D:2 * D], (T, 1))])
    EX = jnp.concatenate([
        jnp.kron(jnp.eye(T, dtype=jnp.float32), jnp.ones((1, D), jnp.float32)),
        jnp.zeros((D - T, T * D), jnp.float32)])

    P = _sc_embed(emb, tok)
    cntP = _sc_pass(OH, cgi, srci)
    GA = _tc_tables(P, WA)
    for step in range(3):
        inP = _sc_pass(GA.reshape(NP * T, D), ga, dsti)
        last = step == 2
        res = _tc_gru(P, inP, cntP, Wr, Wz, Wt, W2, Bs, WB, WA, EX, not last)
        if last:
            P = res[0]
        else:
            P, GA = res
    return P[:N]


# speedup vs baseline: 1.3108x; 1.1086x over previous
import functools

import jax
import jax.numpy as jnp
from jax import lax
from jax.experimental import pallas as pl
from jax.experimental.pallas import tpu as pltpu
from jax.experimental.pallas import tpu_sc as plsc

N = 10000
NP = 10240
E = 160000
EP = 163840
D = 16
T = 8
NC = 2
NS = 16
NW = NC * NS
WIN = 128
NWIN = EP // (NW * WIN)
ROWS_PER_TILE = NP // NW
ZCH = NP // NS

_mesh = plsc.VectorSubcoreMesh(core_axis_name="c", subcore_axis_name="s")
_sc_params = pltpu.CompilerParams(use_tc_tiling_on_sc=False)


@functools.partial(
    pl.kernel,
    out_type=jax.ShapeDtypeStruct((NP, D), jnp.float32),
    mesh=_mesh,
    scratch_types=[
        pltpu.VMEM((4, 80), jnp.int32),
        pltpu.VMEM((80, D), jnp.float32),
    ],
    compiler_params=_sc_params,
)
def _sc_embed(emb_hbm, tok_hbm, out_hbm, idx_v, rows_v):
    c = lax.axis_index("c")
    s = lax.axis_index("s")
    wid = c * NS + s
    pltpu.sync_copy(tok_hbm.at[wid], idx_v)

    @pl.loop(0, 4)
    def _(k):
        pltpu.sync_copy(emb_hbm.at[idx_v.at[k]], rows_v)
        pltpu.sync_copy(rows_v, out_hbm.at[pl.ds(wid * ROWS_PER_TILE + k * 80, 80)])


@functools.partial(
    pl.kernel,
    out_type=jax.ShapeDtypeStruct((NC, NP, D), jnp.float32),
    mesh=_mesh,
    scratch_types=[
        pltpu.VMEM((NWIN, WIN), jnp.int32),
        pltpu.VMEM((NWIN, WIN), jnp.int32),
        pltpu.VMEM((8, WIN, D), jnp.float32),
        pltpu.VMEM((WIN, D), jnp.float32),
        pltpu.VMEM_SHARED((NP, D), jnp.float32),
        pltpu.SemaphoreType.DMA((8,)),
        pltpu.SemaphoreType.DMA((2,)),
    ],
    compiler_params=_sc_params,
)
def _sc_pass(tab, gi_hbm, si_hbm, outP, gv, sv, rows, zv, acc, semG, semI):
    c = lax.axis_index("c")
    s = lax.axis_index("s")
    wid = c * NS + s

    pltpu.async_copy(gi_hbm.at[wid], gv, semI.at[0])
    pltpu.async_copy(si_hbm.at[wid], sv, semI.at[1])

    @pl.loop(0, WIN)
    def _(i):
        zv[i, :] = jnp.zeros((D,), jnp.float32)

    @pl.loop(0, ZCH // WIN)
    def _(k):
        pltpu.sync_copy(zv, acc.at[pl.ds(s * ZCH + k * WIN, WIN)])

    pltpu.make_async_copy(gi_hbm.at[wid], gv, semI.at[0]).wait()
    pltpu.make_async_copy(si_hbm.at[wid], sv, semI.at[1]).wait()

    plsc.subcore_barrier()

    for k in range(8):
        pltpu.async_copy(tab.at[gv.at[k]], rows.at[k], semG.at[k])

    @pl.loop(0, NWIN // 8)
    def _(i):
        for k in range(8):
            w = 8 * i + k
            pltpu.make_async_copy(tab.at[gv.at[w]], rows.at[k], semG.at[k]).wait()
            pltpu.sync_copy(rows.at[k], acc.at[sv.at[w]], add=True)

            @pl.when(w + 8 < NWIN)
            def _():
                pltpu.async_copy(tab.at[gv.at[w + 8]], rows.at[k], semG.at[k])

    plsc.subcore_barrier()

    pltpu.sync_copy(acc.at[pl.ds(s * ZCH, ZCH)], outP.at[c, pl.ds(s * ZCH, ZCH)])


_R = 2048
_NPB = NP // _R


def _tables0_body(p_ref, wa_ref, ga_ref):
    ga_ref[...] = jnp.dot(p_ref[...], wa_ref[...], preferred_element_type=jnp.float32)


def _tc_tables(P, WA):
    return pl.pallas_call(
        _tables0_body,
        grid=(_NPB,),
        in_specs=[
            pl.BlockSpec((_R, D), lambda i: (i, 0)),
            pl.BlockSpec((D, T * D), lambda i: (0, 0)),
        ],
        out_specs=pl.BlockSpec((_R, T * D), lambda i: (i, 0)),
        out_shape=jax.ShapeDtypeStruct((NP, T * D), jnp.float32),
    )(P, WA)


def _gru_body(with_tables, p_ref, inp_ref, cnt_ref, wr_ref, wz_ref, wt_ref,
              w2_ref, b_ref, wb_ref, wa_ref, ex_ref, *out_refs):
    p = p_ref[...]
    infea = inp_ref[0] + inp_ref[1]
    cnt = cnt_ref[0] + cnt_ref[1]
    zb = jnp.dot(p, wb_ref[...], preferred_element_type=jnp.float32)
    owide = jnp.dot(cnt, ex_ref[...], preferred_element_type=jnp.float32) * zb

    def lin(w_ref, w2, x3, brow):
        w = w_ref[...]
        return (jnp.dot(infea, w[0:D], preferred_element_type=jnp.float32)
                + jnp.dot(owide, w2, preferred_element_type=jnp.float32)
                + jnp.dot(x3, w[2 * D:3 * D], preferred_element_type=jnp.float32)
                + brow)

    b = b_ref[...]
    w2 = w2_ref[...]
    r = jax.nn.sigmoid(lin(wr_ref, w2[0], p, b[0:1]))
    z = jax.nn.sigmoid(lin(wz_ref, w2[1], p, b[1:2]))
    h = jnp.tanh(lin(wt_ref, w2[2], r * p, b[2:3]))
    pnew = (1.0 - z) * p + z * h
    out_refs[0][...] = pnew
    if with_tables:
        out_refs[1][...] = jnp.dot(pnew, wa_ref[...], preferred_element_type=jnp.float32)


def _tc_gru(P, inP, cntP, Wr, Wz, Wt, W2, Bs, WB, WA, EX, with_tables):
    out_shape = [jax.ShapeDtypeStruct((NP, D), jnp.float32)]
    out_specs = [pl.BlockSpec((_R, D), lambda i: (i, 0))]
    if with_tables:
        out_shape.append(jax.ShapeDtypeStruct((NP, T * D), jnp.float32))
        out_specs.append(pl.BlockSpec((_R, T * D), lambda i: (i, 0)))
    return pl.pallas_call(
        functools.partial(_gru_body, with_tables),
        grid=(_NPB,),
        in_specs=[
            pl.BlockSpec((_R, D), lambda i: (i, 0)),
            pl.BlockSpec((NC, _R, D), lambda i: (0, i, 0)),
            pl.BlockSpec((NC, _R, D), lambda i: (0, i, 0)),
            pl.BlockSpec((3 * D, D), lambda i: (0, 0)),
            pl.BlockSpec((3 * D, D), lambda i: (0, 0)),
            pl.BlockSpec((3 * D, D), lambda i: (0, 0)),
            pl.BlockSpec((3, T * D, D), lambda i: (0, 0, 0)),
            pl.BlockSpec((3, D), lambda i: (0, 0)),
            pl.BlockSpec((D, T * D), lambda i: (0, 0)),
            pl.BlockSpec((D, T * D), lambda i: (0, 0)),
            pl.BlockSpec((D, T * D), lambda i: (0, 0)),
        ],
        out_specs=out_specs,
        out_shape=out_shape,
    )(P, inP, cntP, Wr, Wz, Wt, W2, Bs, WB, WA, EX)


def kernel(token, edge_index, edge_type, emb, fc_w, Wr, br, Wz, bz, Wt, bt):
    src = edge_index[0].astype(jnp.int32)
    dst = edge_index[1].astype(jnp.int32)
    et = edge_type.astype(jnp.int32)

    zpad = jnp.zeros((EP - E,), jnp.int32)
    npad = jnp.full((EP - E,), N, jnp.int32)
    ga = jnp.concatenate([src * T + et, zpad]).reshape(NW, NWIN, WIN)
    dsti = jnp.concatenate([dst, npad]).reshape(NW, NWIN, WIN)
    win_id = jnp.arange(EP, dtype=jnp.int32) // WIN
    cg = jnp.concatenate([et, zpad]) + win_id * T
    cgi = cg.reshape(NW, NWIN, WIN)
    srci = jnp.concatenate([src, npad]).reshape(NW, NWIN, WIN)

    tok = jnp.concatenate([token.astype(jnp.int32),
                           jnp.zeros((NP - N,), jnp.int32)]).reshape(NW, 4, 80)

    fw = fc_w.reshape(T, D, 2 * D)
    WA = fw[:, :, 0::2].transpose(1, 0, 2).reshape(D, T * D)
    WB = fw[:, :, 1::2].transpose(1, 0, 2).reshape(D, T * D)
    Bs = jnp.stack([br, bz, bt])
    OH = jnp.tile(jnp.eye(T, D, dtype=jnp.float32), (EP // WIN, 1))
    W2 = jnp.stack([jnp.tile(Wr[D:2 * D], (T, 1)),
                    jnp.tile(Wz[D:2 * D], (T, 1)),
                    jnp.tile(Wt[D:2 * D], (T, 1))])
    EX = jnp.concatenate([
        jnp.kron(jnp.eye(T, dtype=jnp.float32), jnp.ones((1, D), jnp.float32)),
        jnp.zeros((D - T, T * D), jnp.float32)])

    P = _sc_embed(emb, tok)
    cntP = _sc_pass(OH, cgi, srci)
    GA = _tc_tables(P, WA)
    for step in range(3):
        inP = _sc_pass(GA.reshape(NP * T, D), ga, dsti)
        last = step == 2
        res = _tc_gru(P, inP, cntP, Wr, Wz, Wt, W2, Bs, WB, WA, EX, not last)
        if last:
            P = res[0]
        else:
            P, GA = res
    return P[:N]
